# Initial kernel scaffold; baseline (speedup 1.0000x reference)
import jax
import jax.numpy as jnp
from jax.experimental import pallas as pl


def kernel(x, W_gate, We, be):
    raise NotImplementedError("write your pallas kernel here")


# speedup vs baseline: 1.1640x; 1.1640x over previous
import functools
import jax
import jax.numpy as jnp
from jax import lax
from jax.experimental import pallas as pl
from jax.experimental.pallas import tpu as pltpu

E = 8
D = 768
N = 2048
BT = 256


def _moe_block(x_ref, wg_ref, we_ref, be_ref, out_ref):
    xb = x_ref[...]
    gating = jnp.dot(xb, wg_ref[...], preferred_element_type=jnp.float32)
    iota = lax.broadcasted_iota(jnp.int32, (BT, E), 1)
    m1 = jnp.max(gating, axis=1, keepdims=True)
    a1 = jnp.min(jnp.where(gating == m1, iota, E), axis=1, keepdims=True)
    neg = jnp.full_like(gating, -jnp.inf)
    g2d = jnp.where(iota == a1, neg, gating)
    m2 = jnp.max(g2d, axis=1, keepdims=True)
    a2 = jnp.min(jnp.where(g2d == m2, iota, E), axis=1, keepdims=True)
    s = jnp.exp(m2 - m1)
    g1 = 1.0 / (1.0 + s)
    g2 = 1.0 - g1
    acc = jnp.zeros((BT, D), jnp.float32)
    for e in range(E):
        ge = jnp.where(a1 == e, g1, 0.0) + jnp.where(a2 == e, g2, 0.0)
        ye = jnp.dot(xb, we_ref[e], preferred_element_type=jnp.float32)
        acc = acc + ge * (ye + be_ref[e][None, :])
    out_ref[...] = acc


def kernel(x, W_gate, We, be):
    grid = (N // BT,)
    return pl.pallas_call(
        _moe_block,
        grid=grid,
        in_specs=[
            pl.BlockSpec((BT, D), lambda i: (i, 0)),
            pl.BlockSpec((D, E), lambda i: (0, 0)),
            pl.BlockSpec((E, D, D), lambda i: (0, 0, 0)),
            pl.BlockSpec((E, D), lambda i: (0, 0)),
        ],
        out_specs=pl.BlockSpec((BT, D), lambda i: (i, 0)),
        out_shape=jax.ShapeDtypeStruct((N, D), jnp.float32),
    )(x, W_gate, We, be)
